# baseline (speedup 1.0000x reference)
import jax
import jax.numpy as jnp
from jax.experimental import pallas as pl

NUM_EXPERTS = 8
GROUP = 1024
CAP = 256


def _router_kernel(x_ref, wr_ref, meta_i_ref, meta_f_ref, aux_ref, *, n_groups):
    g = pl.program_id(0)
    x = x_ref[0]
    logits = jnp.dot(x, wr_ref[...], preferred_element_type=jnp.float32)
    m = jnp.max(logits, axis=1, keepdims=True)
    p = jnp.exp(logits - m)
    gates = p / jnp.sum(p, axis=1, keepdims=True)
    eio = jax.lax.broadcasted_iota(jnp.int32, (GROUP, NUM_EXPERTS), 1)
    idx1 = jnp.argmax(gates, axis=1).astype(jnp.int32)
    v1 = jnp.max(gates, axis=1)
    oh1 = (eio == idx1[:, None]).astype(jnp.float32)
    gates2 = jnp.where(oh1 > 0, -1.0, gates)
    idx2 = jnp.argmax(gates2, axis=1).astype(jnp.int32)
    v2 = jnp.max(gates2, axis=1)
    oh2 = (eio == idx2[:, None]).astype(jnp.float32)
    r = jax.lax.broadcasted_iota(jnp.int32, (GROUP, GROUP), 0)
    c = jax.lax.broadcasted_iota(jnp.int32, (GROUP, GROUP), 1)
    ltri = (c <= r).astype(jnp.float32)
    csum1 = jnp.dot(ltri, oh1, preferred_element_type=jnp.float32)
    csum2 = jnp.dot(ltri, oh2, preferred_element_type=jnp.float32)
    cnt1 = jnp.sum(oh1, axis=0, keepdims=True)
    pos1 = jnp.sum(oh1 * (csum1 - 1.0), axis=1).astype(jnp.int32)
    pos2 = jnp.sum(oh2 * (cnt1 + csum2 - 1.0), axis=1).astype(jnp.int32)
    pos1 = jnp.where(pos1 < CAP, pos1, -1)
    pos2 = jnp.where(pos2 < CAP, pos2, -1)
    meta_i_ref[0] = jnp.concatenate(
        [idx1[None], idx2[None], pos1[None], pos2[None]], axis=0)
    meta_f_ref[0] = jnp.concatenate([v1[None], v2[None]], axis=0)
    imp = jnp.sum(gates, axis=0, keepdims=True)
    mean = jnp.mean(imp, axis=1, keepdims=True)
    var = jnp.mean((imp - mean) ** 2, axis=1, keepdims=True)
    loss_g = (var / (mean + 1e-10) ** 2) * (1.0 / n_groups)

    @pl.when(g == 0)
    def _():
        aux_ref[...] = loss_g

    @pl.when(g > 0)
    def _():
        aux_ref[...] += loss_g


def _mlp_kernel(xb_ref, meta_i_ref, w1_ref, b1_ref, w2_ref, b2_ref, eo_ref):
    e = pl.program_id(0)
    idx1 = meta_i_ref[0, 0, :]
    idx2 = meta_i_ref[0, 1, :]
    pos1 = meta_i_ref[0, 2, :]
    pos2 = meta_i_ref[0, 3, :]
    cio = jax.lax.broadcasted_iota(jnp.int32, (GROUP, CAP), 1)
    d1 = (idx1[:, None] == e) & (pos1[:, None] == cio)
    d2 = (idx2[:, None] == e) & (pos2[:, None] == cio)
    dmat = (d1 | d2).astype(jnp.bfloat16)
    ein = jax.lax.dot_general(
        dmat, xb_ref[0], (((0,), (0,)), ((), ())),
        preferred_element_type=jnp.float32)
    h = jnp.dot(ein.astype(jnp.bfloat16), w1_ref[0],
                preferred_element_type=jnp.float32)
    h = jax.nn.gelu(h + b1_ref[0])
    out = jnp.dot(h.astype(jnp.bfloat16), w2_ref[0],
                  preferred_element_type=jnp.float32)
    eo_ref[0, 0] = (out + b2_ref[0]).astype(jnp.bfloat16)


def _combine_kernel(meta_i_ref, meta_f_ref, eo_ref, out_ref):
    idx1 = meta_i_ref[0, 0, :]
    idx2 = meta_i_ref[0, 1, :]
    pos1 = meta_i_ref[0, 2, :]
    pos2 = meta_i_ref[0, 3, :]
    v1 = meta_f_ref[0, 0, :]
    v2 = meta_f_ref[0, 1, :]
    cio = jax.lax.broadcasted_iota(jnp.int32, (GROUP, CAP), 1)
    acc = jnp.zeros((GROUP, eo_ref.shape[-1]), jnp.float32)
    for e in range(NUM_EXPERTS):
        c1 = jnp.where((idx1[:, None] == e) & (pos1[:, None] == cio),
                       v1[:, None], 0.0)
        c2 = jnp.where((idx2[:, None] == e) & (pos2[:, None] == cio),
                       v2[:, None], 0.0)
        acc += jnp.dot((c1 + c2).astype(jnp.bfloat16), eo_ref[e, 0],
                       preferred_element_type=jnp.float32)
    out_ref[0] = acc


def kernel(inputs, w_router, w1, b1, w2, b2, *, interpret=False):
    B, S, D = inputs.shape
    E = w_router.shape[1]
    F = w1.shape[2]
    NG = (B * S) // GROUP
    x = inputs.reshape(NG, GROUP, D)
    xb = x.astype(jnp.bfloat16)
    w1b = w1.astype(jnp.bfloat16)
    w2b = w2.astype(jnp.bfloat16)
    b1r = b1.reshape(E, 1, F)
    b2r = b2.reshape(E, 1, D)

    import functools
    meta_i, meta_f, aux = pl.pallas_call(
        functools.partial(_router_kernel, n_groups=NG),
        grid=(NG,),
        in_specs=[pl.BlockSpec((1, GROUP, D), lambda g: (g, 0, 0)),
                  pl.BlockSpec((D, E), lambda g: (0, 0))],
        out_specs=[pl.BlockSpec((1, 4, GROUP), lambda g: (g, 0, 0)),
                   pl.BlockSpec((1, 2, GROUP), lambda g: (g, 0, 0)),
                   pl.BlockSpec((1, 1), lambda g: (0, 0))],
        out_shape=[jax.ShapeDtypeStruct((NG, 4, GROUP), jnp.int32),
                   jax.ShapeDtypeStruct((NG, 2, GROUP), jnp.float32),
                   jax.ShapeDtypeStruct((1, 1), jnp.float32)],
        interpret=interpret,
    )(x, w_router)

    eo = pl.pallas_call(
        _mlp_kernel,
        grid=(E, NG),
        in_specs=[pl.BlockSpec((1, GROUP, D), lambda e, g: (g, 0, 0)),
                  pl.BlockSpec((1, 4, GROUP), lambda e, g: (g, 0, 0)),
                  pl.BlockSpec((1, D, F), lambda e, g: (e, 0, 0)),
                  pl.BlockSpec((1, 1, F), lambda e, g: (e, 0, 0)),
                  pl.BlockSpec((1, F, D), lambda e, g: (e, 0, 0)),
                  pl.BlockSpec((1, 1, D), lambda e, g: (e, 0, 0))],
        out_specs=pl.BlockSpec((1, 1, CAP, D), lambda e, g: (e, g, 0, 0)),
        out_shape=jax.ShapeDtypeStruct((E, NG, CAP, D), jnp.bfloat16),
        interpret=interpret,
    )(xb, meta_i, w1b, b1r, w2b, b2r)

    out = pl.pallas_call(
        _combine_kernel,
        grid=(NG,),
        in_specs=[pl.BlockSpec((1, 4, GROUP), lambda g: (g, 0, 0)),
                  pl.BlockSpec((1, 2, GROUP), lambda g: (g, 0, 0)),
                  pl.BlockSpec((E, 1, CAP, D), lambda g: (0, g, 0, 0))],
        out_specs=pl.BlockSpec((1, GROUP, D), lambda g: (g, 0, 0)),
        out_shape=jax.ShapeDtypeStruct((NG, GROUP, D), jnp.float32),
        interpret=interpret,
    )(meta_i, meta_f, eo)

    return out.reshape(B, S, D), aux.reshape(())
